# baseline (speedup 1.0000x reference)
import functools

import jax
import jax.numpy as jnp
from jax import lax
from jax.experimental import pallas as pl
from jax.experimental.pallas import tpu as pltpu
from jax.experimental.pallas import tpu_sc as plsc

N_M = 20000
N_D = 10000
N_A = 20000
N = N_M + N_D + N_A
E = 800000

CHUNK = 128
STREAMS = 6
SUP = CHUNK * STREAMS
N_TILES = 16
SUPS_PER_TILE = 66
E_PAD = SUPS_PER_TILE * SUP * N_TILES
N_CHUNK_ROWS = E_PAD // CHUNK

NROW_PAD = 50048
ROWS_PER_TILE = NROW_PAD // N_TILES
OUT_ROWS_PER_TILE = N // N_TILES
DUMMY_DST = N

S_ALL = 40000
SP = 40960
TAIL_CHUNKS_PER_W = SP // (32 * CHUNK)

_mesh = plsc.VectorSubcoreMesh(core_axis_name="c", subcore_axis_name="s")


@functools.partial(
    pl.kernel,
    out_type=(
        jax.ShapeDtypeStruct((N, 16), jnp.float32),
        jax.ShapeDtypeStruct((N, 16), jnp.float32),
        jax.ShapeDtypeStruct((N, 16), jnp.float32),
        jax.ShapeDtypeStruct((N, 16), jnp.float32),
    ),
    mesh=_mesh,
    scratch_types=[
        pltpu.VMEM_SHARED((NROW_PAD, 16), jnp.float32),
        pltpu.VMEM_SHARED((NROW_PAD, 16), jnp.float32),
        pltpu.VMEM((STREAMS, CHUNK), jnp.int32),
        pltpu.VMEM((STREAMS, CHUNK), jnp.int32),
        pltpu.VMEM((SUP, 16), jnp.float32),
        pltpu.SemaphoreType.DMA,
        pltpu.SemaphoreType.DMA,
    ],
    compiler_params=pltpu.CompilerParams(use_tc_tiling_on_sc=False),
)
def _sc_agg(src2, dst2, q0, q1, q2, q3, zblk, o0, o1, o2, o3,
            srcbuf, acc, srcv, dstv, rows, gsem, ssem):
    c = lax.axis_index("c")
    s = lax.axis_index("s")

    def run(h_hbm, out_hbm):
        pltpu.sync_copy(h_hbm.at[pl.ds(s * OUT_ROWS_PER_TILE, OUT_ROWS_PER_TILE)],
                        srcbuf.at[pl.ds(s * OUT_ROWS_PER_TILE, OUT_ROWS_PER_TILE)])
        pltpu.sync_copy(zblk, acc.at[pl.ds(s * ROWS_PER_TILE, ROWS_PER_TILE)])
        plsc.subcore_barrier()

        def body(j, carry):
            sup = s * SUPS_PER_TILE + j
            pltpu.sync_copy(src2.at[pl.ds(sup * STREAMS, STREAMS)], srcv)
            pltpu.sync_copy(dst2.at[pl.ds(sup * STREAMS, STREAMS)], dstv)
            descs = []
            for t in range(STREAMS):
                descs.append(pltpu.async_copy(
                    srcbuf.at[srcv.at[t]],
                    rows.at[pl.ds(t * CHUNK, CHUNK)], gsem))
            for d in descs:
                d.wait()
            descs = []
            for t in range(STREAMS):
                descs.append(pltpu.async_copy(
                    rows.at[pl.ds(t * CHUNK, CHUNK)],
                    acc.at[dstv.at[t]], ssem, add=True))
            for d in descs:
                d.wait()
            return carry

        lax.fori_loop(0, SUPS_PER_TILE, body, 0)
        plsc.subcore_barrier()
        pltpu.sync_copy(acc.at[pl.ds(s * OUT_ROWS_PER_TILE, OUT_ROWS_PER_TILE)],
                        out_hbm.at[pl.ds(s * OUT_ROWS_PER_TILE, OUT_ROWS_PER_TILE)])

    @pl.when(c == 0)
    def _():
        run(q0, o0)
        run(q1, o1)

    @pl.when(c == 1)
    def _():
        run(q2, o2)
        run(q3, o3)


@functools.partial(
    pl.kernel,
    out_type=jax.ShapeDtypeStruct((SP, 16), jnp.float32),
    mesh=_mesh,
    scratch_types=[
        pltpu.VMEM((CHUNK,), jnp.int32),
        pltpu.VMEM((CHUNK,), jnp.int32),
        pltpu.VMEM((CHUNK,), jnp.int32),
        pltpu.VMEM((CHUNK, 16), jnp.float32),
        pltpu.VMEM((CHUNK, 16), jnp.float32),
        pltpu.VMEM((CHUNK, 16), jnp.float32),
        pltpu.VMEM((CHUNK, 16), jnp.float32),
        pltpu.SemaphoreType.DMA,
    ],
    compiler_params=pltpu.CompilerParams(use_tc_tiling_on_sc=False),
)
def _sc_tail(abc_hbm, im_hbm, id_hbm, ia_hbm, v_hbm,
             imv, idv, iav, ra, rb, rc, vb, sem):
    c = lax.axis_index("c")
    s = lax.axis_index("s")
    w = s * 2 + c

    def body(j, carry):
        base = (w * TAIL_CHUNKS_PER_W + j) * CHUNK
        pltpu.sync_copy(im_hbm.at[pl.ds(base, CHUNK)], imv)
        pltpu.sync_copy(id_hbm.at[pl.ds(base, CHUNK)], idv)
        pltpu.sync_copy(ia_hbm.at[pl.ds(base, CHUNK)], iav)
        for u in range(CHUNK // 16):
            sl = pl.ds(u * 16, 16)
            idv[sl] = idv[sl] + N_M
            iav[sl] = iav[sl] + (N_M + N_D)
        d1 = pltpu.async_copy(abc_hbm.at[imv], ra, sem)
        d2 = pltpu.async_copy(abc_hbm.at[idv], rb, sem)
        d3 = pltpu.async_copy(abc_hbm.at[iav], rc, sem)
        d1.wait()
        d2.wait()
        d3.wait()

        def inner(i, carry2):
            vb[i, :] = jnp.maximum(ra[i, :] + rb[i, :] + rc[i, :], 0.0)
            return carry2

        lax.fori_loop(0, CHUNK, inner, 0)
        pltpu.sync_copy(vb, v_hbm.at[pl.ds(base, CHUNK)])
        return carry

    lax.fori_loop(0, TAIL_CHUNKS_PER_W, body, 0)


_RB = 2000


def _proj_fused(h_movie, h_director, h_actor, wpm, wpd, wpa, w1):
    nb_m = N_M // _RB
    nb_md = (N_M + N_D) // _RB

    def kern(xm_ref, xd_ref, xa_ref, wpm_ref, wpd_ref, wpa_ref, w1_ref,
             q0_ref, q1_ref, q2_ref, q3_ref):
        i = pl.program_id(0)
        w1 = w1_ref[...]
        ym = jnp.dot(xm_ref[...], jnp.dot(wpm_ref[...], w1,
                                          preferred_element_type=jnp.float32),
                     preferred_element_type=jnp.float32)
        wda = jnp.where(i < nb_md, wpd_ref[...], wpa_ref[...])
        xda = jnp.where(i < nb_md, xd_ref[...], xa_ref[...])
        yda = jnp.dot(xda, jnp.dot(wda, w1, preferred_element_type=jnp.float32),
                      preferred_element_type=jnp.float32)
        y = jnp.where(i < nb_m, ym, yda)
        q0_ref[...] = y[:, :16]
        q1_ref[...] = y[:, 16:32]
        q2_ref[...] = y[:, 32:48]
        q3_ref[...] = y[:, 48:]

    return pl.pallas_call(
        kern,
        grid=(N // _RB,),
        in_specs=[
            pl.BlockSpec((_RB, 128), lambda i: (jnp.minimum(i, nb_m - 1), 0)),
            pl.BlockSpec((_RB, 64),
                         lambda i: (jnp.clip(i - nb_m, 0, N_D // _RB - 1), 0)),
            pl.BlockSpec((_RB, 64),
                         lambda i: (jnp.clip(i - nb_md, 0, N_A // _RB - 1), 0)),
            pl.BlockSpec((128, 64), lambda i: (0, 0)),
            pl.BlockSpec((64, 64), lambda i: (0, 0)),
            pl.BlockSpec((64, 64), lambda i: (0, 0)),
            pl.BlockSpec((64, 64), lambda i: (0, 0)),
        ],
        out_specs=[pl.BlockSpec((_RB, 16), lambda i: (i, 0))] * 4,
        out_shape=tuple(jax.ShapeDtypeStruct((N, 16), jnp.float32)
                        for _ in range(4)),
    )(h_movie, h_director, h_actor, wpm, wpd, wpa, w1)


def _relu_mm_split(qs, b1, w2):
    def kern(q0_ref, q1_ref, q2_ref, q3_ref, b_ref, w_ref,
             o0_ref, o1_ref, o2_ref, o3_ref):
        x = jnp.concatenate([q0_ref[...], q1_ref[...],
                             q2_ref[...], q3_ref[...]], axis=1)
        t = jnp.maximum(x + b_ref[...], 0.0)
        y = jnp.dot(t, w_ref[...], preferred_element_type=jnp.float32)
        o0_ref[...] = y[:, :16]
        o1_ref[...] = y[:, 16:32]
        o2_ref[...] = y[:, 32:48]
        o3_ref[...] = y[:, 48:]

    return pl.pallas_call(
        kern,
        grid=(N // _RB,),
        in_specs=[pl.BlockSpec((_RB, 16), lambda i: (i, 0))] * 4 + [
                  pl.BlockSpec((1, 64), lambda i: (0, 0)),
                  pl.BlockSpec((64, 64), lambda i: (0, 0))],
        out_specs=[pl.BlockSpec((_RB, 16), lambda i: (i, 0))] * 4,
        out_shape=tuple(jax.ShapeDtypeStruct((N, 16), jnp.float32)
                        for _ in range(4)),
    )(*qs, b1.reshape(1, 64), w2)


def _norm_abc(qs, b2, w_ctx_d, w_ctx_a, w_hid):
    nb_m = N_M // _RB
    nb_md = (N_M + N_D) // _RB

    def kern(q0_ref, q1_ref, q2_ref, q3_ref, b_ref, wd_ref, wa_ref, wh_ref,
             h_ref, abc_ref):
        i = pl.program_id(0)
        x = jnp.concatenate([q0_ref[...], q1_ref[...],
                             q2_ref[...], q3_ref[...]], axis=1)
        t = x + b_ref[...]
        n = jnp.sqrt(jnp.sum(t * t, axis=1, keepdims=True))
        h = t / jnp.maximum(n, 1e-12)
        h_ref[...] = h
        wh = wh_ref[...]
        wm = wh[:64, :]
        wd = jnp.dot(wd_ref[...], wh[64:80, :], preferred_element_type=jnp.float32)
        wa = jnp.dot(wa_ref[...], wh[80:96, :], preferred_element_type=jnp.float32)
        w = jnp.where(i < nb_m, wm, jnp.where(i < nb_md, wd, wa))
        abc = jnp.dot(h, w, preferred_element_type=jnp.float32)
        abc_ref[...] = abc

    return pl.pallas_call(
        kern,
        grid=(N // _RB,),
        in_specs=[pl.BlockSpec((_RB, 16), lambda i: (i, 0))] * 4 + [
                  pl.BlockSpec((1, 64), lambda i: (0, 0)),
                  pl.BlockSpec((64, 16), lambda i: (0, 0)),
                  pl.BlockSpec((64, 16), lambda i: (0, 0)),
                  pl.BlockSpec((96, 16), lambda i: (0, 0))],
        out_specs=[pl.BlockSpec((_RB, 64), lambda i: (i, 0)),
                   pl.BlockSpec((_RB, 16), lambda i: (i, 0))],
        out_shape=(jax.ShapeDtypeStruct((N, 64), jnp.float32),
                   jax.ShapeDtypeStruct((N, 16), jnp.float32)),
    )(*qs, b2.reshape(1, 64), w_ctx_d, w_ctx_a, w_hid)


def _sigmoid_dot(v, w_out, rblk=2048):
    def kern(v_ref, w_ref, o_ref):
        p = jnp.sum(v_ref[...] * w_ref[...], axis=1)
        o_ref[...] = jax.nn.sigmoid(p).reshape(rblk // 128, 128)

    return pl.pallas_call(
        kern,
        grid=(SP // rblk,),
        in_specs=[pl.BlockSpec((rblk, 16), lambda i: (i, 0)),
                  pl.BlockSpec((1, 16), lambda i: (0, 0))],
        out_specs=pl.BlockSpec((rblk // 128, 128), lambda i: (i, 0)),
        out_shape=jax.ShapeDtypeStruct((SP // 128, 128), jnp.float32),
    )(v, w_out.reshape(1, 16))


def kernel(h_movie, h_director, h_actor, edge_index, ns1_movie, ns1_director,
           ns1_actor, ns2_movie, ns2_director, ns2_actor, W_proj_movie,
           W_proj_director, W_proj_actor, W_gnn1, b_gnn1, W_gnn2, b_gnn2,
           W_ctx_movie, W_ctx_director, W_ctx_actor, W_hid, W_out):
    q = _proj_fused(h_movie, h_director, h_actor,
                    W_proj_movie, W_proj_director, W_proj_actor, W_gnn1)

    npad = E_PAD - E
    src2 = jnp.concatenate(
        [edge_index[0], jnp.zeros((npad,), jnp.int32)]).reshape(N_CHUNK_ROWS, CHUNK)
    dst2 = jnp.concatenate(
        [edge_index[1], jnp.full((npad,), DUMMY_DST, jnp.int32)]).reshape(N_CHUNK_ROWS, CHUNK)
    zblk = jnp.zeros((ROWS_PER_TILE, 16), jnp.float32)

    a1 = _sc_agg(src2, dst2, *q, zblk)
    h1 = _relu_mm_split(a1, b_gnn1, W_gnn2)
    a2 = _sc_agg(src2, dst2, *h1, zblk)
    h, abc = _norm_abc(a2, b_gnn2, W_ctx_director, W_ctx_actor, W_hid)

    spad = SP - S_ALL
    zpad = jnp.zeros((spad,), jnp.int32)
    im = jnp.concatenate([ns1_movie, ns2_movie, zpad])
    idd = jnp.concatenate([ns1_director, ns2_director, zpad])
    ia = jnp.concatenate([ns1_actor, ns2_actor, zpad])

    v = _sc_tail(abc, im, idd, ia)
    x = _sigmoid_dot(v, W_out).reshape(SP)[:S_ALL]

    out_h = h[4353:8029]
    return (h, x, out_h)


# speedup vs baseline: 1.5724x; 1.5724x over previous
import functools

import jax
import jax.numpy as jnp
from jax import lax
from jax.experimental import pallas as pl
from jax.experimental.pallas import tpu as pltpu
from jax.experimental.pallas import tpu_sc as plsc

N_M = 20000
N_D = 10000
N_A = 20000
N = N_M + N_D + N_A
E = 800000

CHUNK = 128
STREAMS = 6
N_TILES = 16
N_STREAMS = E // CHUNK
NS_BASE = N_STREAMS // N_TILES
BURSTS = NS_BASE // STREAMS

NROW_PAD = 50048
ROWS_PER_TILE = NROW_PAD // N_TILES
OUT_ROWS_PER_TILE = N // N_TILES

S_ALL = 40000
SP = 40960
TAIL_CHUNKS_PER_W = SP // (32 * CHUNK)

_mesh = plsc.VectorSubcoreMesh(core_axis_name="c", subcore_axis_name="s")


@functools.partial(
    pl.kernel,
    out_type=(
        jax.ShapeDtypeStruct((N, 32), jnp.float32),
        jax.ShapeDtypeStruct((N, 32), jnp.float32),
    ),
    mesh=_mesh,
    scratch_types=[
        pltpu.VMEM_SHARED((NROW_PAD, 32), jnp.float32),
        pltpu.VMEM((STREAMS, CHUNK), jnp.int32),
        pltpu.VMEM((STREAMS, CHUNK), jnp.int32),
        pltpu.VMEM((STREAMS * CHUNK, 32), jnp.float32),
        pltpu.SemaphoreType.DMA,
        pltpu.SemaphoreType.DMA,
        pltpu.SemaphoreType.DMA,
    ],
    compiler_params=pltpu.CompilerParams(use_tc_tiling_on_sc=False),
)
def _sc_agg(ei, h_lo, h_hi, zblk, out_lo, out_hi,
            acc, srcv, dstv, rows, isem, gsem, ssem):
    c = lax.axis_index("c")
    s = lax.axis_index("s")

    pltpu.sync_copy(zblk, acc.at[pl.ds(s * ROWS_PER_TILE, ROWS_PER_TILE)])
    plsc.subcore_barrier()

    first = s * NS_BASE + jnp.minimum(s, 10)
    extra = s < 10

    def run(h_hbm, out_hbm):
        def do_streams(base_stream, trange):
            descs = []
            for t in trange:
                cb = (base_stream + t) * CHUNK
                descs.append(pltpu.async_copy(
                    ei.at[0, pl.ds(cb, CHUNK)], srcv.at[t], isem))
                descs.append(pltpu.async_copy(
                    ei.at[1, pl.ds(cb, CHUNK)], dstv.at[t], isem))
            for d in descs:
                d.wait()
            descs = []
            for t in trange:
                descs.append(pltpu.async_copy(
                    h_hbm.at[srcv.at[t]],
                    rows.at[pl.ds(t * CHUNK, CHUNK)], gsem))
            for d in descs:
                d.wait()
            descs = []
            for t in trange:
                descs.append(pltpu.async_copy(
                    rows.at[pl.ds(t * CHUNK, CHUNK)],
                    acc.at[dstv.at[t]], ssem, add=True))
            for d in descs:
                d.wait()

        def body(j, carry):
            do_streams(first + j * STREAMS, range(STREAMS))
            return carry

        lax.fori_loop(0, BURSTS, body, 0)

        @pl.when(extra)
        def _():
            do_streams(first + NS_BASE, range(1))

        plsc.subcore_barrier()
        pltpu.sync_copy(acc.at[pl.ds(s * OUT_ROWS_PER_TILE, OUT_ROWS_PER_TILE)],
                        out_hbm.at[pl.ds(s * OUT_ROWS_PER_TILE, OUT_ROWS_PER_TILE)])

    @pl.when(c == 0)
    def _():
        run(h_lo, out_lo)

    @pl.when(c == 1)
    def _():
        run(h_hi, out_hi)


@functools.partial(
    pl.kernel,
    out_type=jax.ShapeDtypeStruct((SP,), jnp.float32),
    mesh=_mesh,
    scratch_types=[
        pltpu.VMEM((CHUNK,), jnp.int32),
        pltpu.VMEM((CHUNK,), jnp.int32),
        pltpu.VMEM((CHUNK,), jnp.int32),
        pltpu.VMEM((CHUNK, 16), jnp.float32),
        pltpu.VMEM((CHUNK, 16), jnp.float32),
        pltpu.VMEM((CHUNK, 16), jnp.float32),
        pltpu.VMEM((CHUNK, 16), jnp.float32),
        pltpu.VMEM((CHUNK,), jnp.float32),
        pltpu.VMEM((16,), jnp.float32),
        pltpu.SemaphoreType.DMA,
    ],
    compiler_params=pltpu.CompilerParams(use_tc_tiling_on_sc=False,
                                         needs_layout_passes=False),
)
def _sc_tail(abc_hbm, im_hbm, id_hbm, ia_hbm, w_hbm, x_hbm,
             imv, idv, iav, ra, rb, rc, tb, xv, wbuf, sem):
    c = lax.axis_index("c")
    s = lax.axis_index("s")
    w = s * 2 + c

    pltpu.sync_copy(w_hbm, wbuf)
    wv = wbuf[...]

    def body(j, carry):
        base = (w * TAIL_CHUNKS_PER_W + j) * CHUNK
        pltpu.sync_copy(im_hbm.at[pl.ds(base, CHUNK)], imv)
        pltpu.sync_copy(id_hbm.at[pl.ds(base, CHUNK)], idv)
        pltpu.sync_copy(ia_hbm.at[pl.ds(base, CHUNK)], iav)
        for u in range(CHUNK // 16):
            sl = pl.ds(u * 16, 16)
            idv[sl] = idv[sl] + N_M
            iav[sl] = iav[sl] + (N_M + N_D)
        d1 = pltpu.async_copy(abc_hbm.at[imv], ra, sem)
        d2 = pltpu.async_copy(abc_hbm.at[idv], rb, sem)
        d3 = pltpu.async_copy(abc_hbm.at[iav], rc, sem)
        d1.wait()
        d2.wait()
        d3.wait()

        def inner(i, carry2):
            row = jnp.maximum(ra[i, :] + rb[i, :] + rc[i, :], 0.0)
            tb[i, :] = row * wv
            return carry2

        lax.fori_loop(0, CHUNK, inner, 0)
        for u in range(CHUNK // 16):
            rowid = jax.lax.iota(jnp.int32, 16) + (u * 16)
            p = jnp.zeros((16,), jnp.float32)
            for f in range(16):
                p = p + plsc.load_gather(
                    tb, [rowid, jnp.full((16,), f, jnp.int32)])
            xv[pl.ds(u * 16, 16)] = 1.0 / (1.0 + jnp.exp(-p))
        pltpu.sync_copy(xv, x_hbm.at[pl.ds(base, CHUNK)])
        return carry

    lax.fori_loop(0, TAIL_CHUNKS_PER_W, body, 0)


_RB = 2000


def _proj_fused(h_movie, h_director, h_actor, wpm, wpd, wpa, w1):
    nb_m = N_M // _RB
    nb_md = (N_M + N_D) // _RB

    def kern(xm_ref, xd_ref, xa_ref, wpm_ref, wpd_ref, wpa_ref, w1_ref,
             lo_ref, hi_ref):
        i = pl.program_id(0)
        w1 = w1_ref[...]
        ym = jnp.dot(xm_ref[...], jnp.dot(wpm_ref[...], w1,
                                          preferred_element_type=jnp.float32),
                     preferred_element_type=jnp.float32)
        wda = jnp.where(i < nb_md, wpd_ref[...], wpa_ref[...])
        xda = jnp.where(i < nb_md, xd_ref[...], xa_ref[...])
        yda = jnp.dot(xda, jnp.dot(wda, w1, preferred_element_type=jnp.float32),
                      preferred_element_type=jnp.float32)
        y = jnp.where(i < nb_m, ym, yda)
        lo_ref[...] = y[:, :32]
        hi_ref[...] = y[:, 32:]

    return pl.pallas_call(
        kern,
        grid=(N // _RB,),
        in_specs=[
            pl.BlockSpec((_RB, 128), lambda i: (jnp.minimum(i, nb_m - 1), 0)),
            pl.BlockSpec((_RB, 64),
                         lambda i: (jnp.clip(i - nb_m, 0, N_D // _RB - 1), 0)),
            pl.BlockSpec((_RB, 64),
                         lambda i: (jnp.clip(i - nb_md, 0, N_A // _RB - 1), 0)),
            pl.BlockSpec((128, 64), lambda i: (0, 0)),
            pl.BlockSpec((64, 64), lambda i: (0, 0)),
            pl.BlockSpec((64, 64), lambda i: (0, 0)),
            pl.BlockSpec((64, 64), lambda i: (0, 0)),
        ],
        out_specs=[pl.BlockSpec((_RB, 32), lambda i: (i, 0)),
                   pl.BlockSpec((_RB, 32), lambda i: (i, 0))],
        out_shape=(jax.ShapeDtypeStruct((N, 32), jnp.float32),
                   jax.ShapeDtypeStruct((N, 32), jnp.float32)),
    )(h_movie, h_director, h_actor, wpm, wpd, wpa, w1)


def _relu_mm_split(lo, hi, b1, w2):
    def kern(lo_ref, hi_ref, b_ref, w_ref, olo_ref, ohi_ref):
        x = jnp.concatenate([lo_ref[...], hi_ref[...]], axis=1)
        t = jnp.maximum(x + b_ref[...], 0.0)
        y = jnp.dot(t, w_ref[...], preferred_element_type=jnp.float32)
        olo_ref[...] = y[:, :32]
        ohi_ref[...] = y[:, 32:]

    return pl.pallas_call(
        kern,
        grid=(N // _RB,),
        in_specs=[pl.BlockSpec((_RB, 32), lambda i: (i, 0)),
                  pl.BlockSpec((_RB, 32), lambda i: (i, 0)),
                  pl.BlockSpec((1, 64), lambda i: (0, 0)),
                  pl.BlockSpec((64, 64), lambda i: (0, 0))],
        out_specs=[pl.BlockSpec((_RB, 32), lambda i: (i, 0)),
                   pl.BlockSpec((_RB, 32), lambda i: (i, 0))],
        out_shape=(jax.ShapeDtypeStruct((N, 32), jnp.float32),
                   jax.ShapeDtypeStruct((N, 32), jnp.float32)),
    )(lo, hi, b1.reshape(1, 64), w2)


def _norm_abc(lo, hi, b2, w_ctx_d, w_ctx_a, w_hid):
    nb_m = N_M // _RB
    nb_md = (N_M + N_D) // _RB

    def kern(lo_ref, hi_ref, b_ref, wd_ref, wa_ref, wh_ref, h_ref, abc_ref):
        i = pl.program_id(0)
        x = jnp.concatenate([lo_ref[...], hi_ref[...]], axis=1)
        t = x + b_ref[...]
        n = jnp.sqrt(jnp.sum(t * t, axis=1, keepdims=True))
        h = t / jnp.maximum(n, 1e-12)
        h_ref[...] = h
        wh = wh_ref[...]
        wm = wh[:64, :]
        wd = jnp.dot(wd_ref[...], wh[64:80, :], preferred_element_type=jnp.float32)
        wa = jnp.dot(wa_ref[...], wh[80:96, :], preferred_element_type=jnp.float32)
        w = jnp.where(i < nb_m, wm, jnp.where(i < nb_md, wd, wa))
        abc_ref[...] = jnp.dot(h, w, preferred_element_type=jnp.float32)

    return pl.pallas_call(
        kern,
        grid=(N // _RB,),
        in_specs=[pl.BlockSpec((_RB, 32), lambda i: (i, 0)),
                  pl.BlockSpec((_RB, 32), lambda i: (i, 0)),
                  pl.BlockSpec((1, 64), lambda i: (0, 0)),
                  pl.BlockSpec((64, 16), lambda i: (0, 0)),
                  pl.BlockSpec((64, 16), lambda i: (0, 0)),
                  pl.BlockSpec((96, 16), lambda i: (0, 0))],
        out_specs=[pl.BlockSpec((_RB, 64), lambda i: (i, 0)),
                   pl.BlockSpec((_RB, 16), lambda i: (i, 0))],
        out_shape=(jax.ShapeDtypeStruct((N, 64), jnp.float32),
                   jax.ShapeDtypeStruct((N, 16), jnp.float32)),
    )(lo, hi, b2.reshape(1, 64), w_ctx_d, w_ctx_a, w_hid)


def kernel(h_movie, h_director, h_actor, edge_index, ns1_movie, ns1_director,
           ns1_actor, ns2_movie, ns2_director, ns2_actor, W_proj_movie,
           W_proj_director, W_proj_actor, W_gnn1, b_gnn1, W_gnn2, b_gnn2,
           W_ctx_movie, W_ctx_director, W_ctx_actor, W_hid, W_out):
    lo0, hi0 = _proj_fused(h_movie, h_director, h_actor,
                           W_proj_movie, W_proj_director, W_proj_actor, W_gnn1)

    zblk = jnp.zeros((ROWS_PER_TILE, 32), jnp.float32)

    a1lo, a1hi = _sc_agg(edge_index, lo0, hi0, zblk)
    h1lo, h1hi = _relu_mm_split(a1lo, a1hi, b_gnn1, W_gnn2)
    a2lo, a2hi = _sc_agg(edge_index, h1lo, h1hi, zblk)
    h, abc = _norm_abc(a2lo, a2hi, b_gnn2, W_ctx_director, W_ctx_actor, W_hid)

    spad = SP - S_ALL
    zpad = jnp.zeros((spad,), jnp.int32)
    im = jnp.concatenate([ns1_movie, ns2_movie, zpad])
    idd = jnp.concatenate([ns1_director, ns2_director, zpad])
    ia = jnp.concatenate([ns1_actor, ns2_actor, zpad])

    x_full = _sc_tail(abc, im, idd, ia, W_out.reshape(16))
    x = x_full[:S_ALL]

    out_h = h[4353:8029]
    return (h, x, out_h)


# speedup vs baseline: 1.7057x; 1.0848x over previous
import functools

import jax
import jax.numpy as jnp
from jax import lax
from jax.experimental import pallas as pl
from jax.experimental.pallas import tpu as pltpu
from jax.experimental.pallas import tpu_sc as plsc

N_M = 20000
N_D = 10000
N_A = 20000
N = N_M + N_D + N_A
E = 800000

CHUNK = 128
STREAMS = 6
N_TILES = 16
N_STREAMS = E // CHUNK
NS_BASE = N_STREAMS // N_TILES
BURSTS = NS_BASE // STREAMS

NROW_PAD = 50048
ROWS_PER_TILE = NROW_PAD // N_TILES
OUT_ROWS_PER_TILE = N // N_TILES

S_ALL = 40000
SP = 40960
TAIL_CHUNKS_PER_W = SP // (32 * CHUNK)

_mesh = plsc.VectorSubcoreMesh(core_axis_name="c", subcore_axis_name="s")


@functools.partial(
    pl.kernel,
    out_type=(
        jax.ShapeDtypeStruct((N, 32), jnp.float32),
        jax.ShapeDtypeStruct((N, 32), jnp.float32),
    ),
    mesh=_mesh,
    scratch_types=[
        pltpu.VMEM_SHARED((NROW_PAD, 32), jnp.float32),
        pltpu.VMEM((STREAMS, CHUNK), jnp.int32),
        pltpu.VMEM((STREAMS, CHUNK), jnp.int32),
        pltpu.VMEM((STREAMS * CHUNK, 32), jnp.float32),
        pltpu.SemaphoreType.DMA,
        pltpu.SemaphoreType.DMA,
        pltpu.SemaphoreType.DMA,
    ],
    compiler_params=pltpu.CompilerParams(use_tc_tiling_on_sc=False),
)
def _sc_agg(ei, h_lo, h_hi, zblk, out_lo, out_hi,
            acc, srcv, dstv, rows, isem, gsem, ssem):
    c = lax.axis_index("c")
    s = lax.axis_index("s")

    pltpu.sync_copy(zblk, acc.at[pl.ds(s * ROWS_PER_TILE, ROWS_PER_TILE)])
    plsc.subcore_barrier()

    first = s * NS_BASE + jnp.minimum(s, 10)
    extra = s < 10

    def run(h_hbm, out_hbm):
        def do_streams(base_stream, trange):
            descs = []
            for t in trange:
                cb = (base_stream + t) * CHUNK
                descs.append(pltpu.async_copy(
                    ei.at[0, pl.ds(cb, CHUNK)], srcv.at[t], isem))
                descs.append(pltpu.async_copy(
                    ei.at[1, pl.ds(cb, CHUNK)], dstv.at[t], isem))
            for d in descs:
                d.wait()
            descs = []
            for t in trange:
                descs.append(pltpu.async_copy(
                    h_hbm.at[srcv.at[t]],
                    rows.at[pl.ds(t * CHUNK, CHUNK)], gsem))
            for d in descs:
                d.wait()
            descs = []
            for t in trange:
                descs.append(pltpu.async_copy(
                    rows.at[pl.ds(t * CHUNK, CHUNK)],
                    acc.at[dstv.at[t]], ssem, add=True))
            for d in descs:
                d.wait()

        def body(j, carry):
            do_streams(first + j * STREAMS, range(STREAMS))
            return carry

        lax.fori_loop(0, BURSTS, body, 0)

        @pl.when(extra)
        def _():
            do_streams(first + NS_BASE, range(1))

        plsc.subcore_barrier()
        pltpu.sync_copy(acc.at[pl.ds(s * OUT_ROWS_PER_TILE, OUT_ROWS_PER_TILE)],
                        out_hbm.at[pl.ds(s * OUT_ROWS_PER_TILE, OUT_ROWS_PER_TILE)])

    @pl.when(c == 0)
    def _():
        run(h_lo, out_lo)

    @pl.when(c == 1)
    def _():
        run(h_hi, out_hi)


@functools.partial(
    pl.kernel,
    out_type=jax.ShapeDtypeStruct((SP,), jnp.float32),
    mesh=_mesh,
    scratch_types=[
        pltpu.VMEM((CHUNK,), jnp.int32),
        pltpu.VMEM((CHUNK,), jnp.int32),
        pltpu.VMEM((CHUNK,), jnp.int32),
        pltpu.VMEM((CHUNK, 16), jnp.float32),
        pltpu.VMEM((CHUNK, 16), jnp.float32),
        pltpu.VMEM((CHUNK, 16), jnp.float32),
        pltpu.VMEM((CHUNK, 16), jnp.float32),
        pltpu.VMEM((CHUNK,), jnp.float32),
        pltpu.VMEM((16,), jnp.float32),
        pltpu.SemaphoreType.DMA,
    ],
    compiler_params=pltpu.CompilerParams(use_tc_tiling_on_sc=False,
                                         needs_layout_passes=False),
)
def _sc_tail(abc_hbm, im_hbm, id_hbm, ia_hbm, w_hbm, x_hbm,
             imv, idv, iav, ra, rb, rc, tb, xv, wbuf, sem):
    c = lax.axis_index("c")
    s = lax.axis_index("s")
    w = s * 2 + c

    pltpu.sync_copy(w_hbm, wbuf)
    wv = wbuf[...]

    def body(j, carry):
        base = (w * TAIL_CHUNKS_PER_W + j) * CHUNK
        pltpu.sync_copy(im_hbm.at[pl.ds(base, CHUNK)], imv)
        pltpu.sync_copy(id_hbm.at[pl.ds(base, CHUNK)], idv)
        pltpu.sync_copy(ia_hbm.at[pl.ds(base, CHUNK)], iav)
        for u in range(CHUNK // 16):
            sl = pl.ds(u * 16, 16)
            idv[sl] = idv[sl] + N_M
            iav[sl] = iav[sl] + (N_M + N_D)
        d1 = pltpu.async_copy(abc_hbm.at[imv], ra, sem)
        d2 = pltpu.async_copy(abc_hbm.at[idv], rb, sem)
        d3 = pltpu.async_copy(abc_hbm.at[iav], rc, sem)
        d1.wait()
        d2.wait()
        d3.wait()

        def inner(i, carry2):
            row = jnp.maximum(ra[i, :] + rb[i, :] + rc[i, :], 0.0)
            tb[i, :] = row * wv
            return carry2

        lax.fori_loop(0, CHUNK, inner, 0)
        for u in range(CHUNK // 16):
            rowid = jax.lax.iota(jnp.int32, 16) + (u * 16)
            p = jnp.zeros((16,), jnp.float32)
            for f in range(16):
                p = p + plsc.load_gather(
                    tb, [rowid, jnp.full((16,), f, jnp.int32)])
            xv[pl.ds(u * 16, 16)] = 1.0 / (1.0 + jnp.exp(-p))
        pltpu.sync_copy(xv, x_hbm.at[pl.ds(base, CHUNK)])
        return carry

    lax.fori_loop(0, TAIL_CHUNKS_PER_W, body, 0)


_RB = 2000


def _proj_fused(h_movie, h_director, h_actor, wpm, wpd, wpa, w1):
    nb_m = N_M // _RB
    nb_md = (N_M + N_D) // _RB

    def kern(xm_ref, xd_ref, xa_ref, wpm_ref, wpd_ref, wpa_ref, w1_ref,
             lo_ref, hi_ref):
        i = pl.program_id(0)
        w1 = w1_ref[...]
        ym = jnp.dot(xm_ref[...], jnp.dot(wpm_ref[...], w1,
                                          preferred_element_type=jnp.float32),
                     preferred_element_type=jnp.float32)
        wda = jnp.where(i < nb_md, wpd_ref[...], wpa_ref[...])
        xda = jnp.where(i < nb_md, xd_ref[...], xa_ref[...])
        yda = jnp.dot(xda, jnp.dot(wda, w1, preferred_element_type=jnp.float32),
                      preferred_element_type=jnp.float32)
        y = jnp.where(i < nb_m, ym, yda)
        lo_ref[...] = y[:, :32]
        hi_ref[...] = y[:, 32:]

    return pl.pallas_call(
        kern,
        grid=(N // _RB,),
        in_specs=[
            pl.BlockSpec((_RB, 128), lambda i: (jnp.minimum(i, nb_m - 1), 0)),
            pl.BlockSpec((_RB, 64),
                         lambda i: (jnp.clip(i - nb_m, 0, N_D // _RB - 1), 0)),
            pl.BlockSpec((_RB, 64),
                         lambda i: (jnp.clip(i - nb_md, 0, N_A // _RB - 1), 0)),
            pl.BlockSpec((128, 64), lambda i: (0, 0)),
            pl.BlockSpec((64, 64), lambda i: (0, 0)),
            pl.BlockSpec((64, 64), lambda i: (0, 0)),
            pl.BlockSpec((64, 64), lambda i: (0, 0)),
        ],
        out_specs=[pl.BlockSpec((_RB, 32), lambda i: (i, 0)),
                   pl.BlockSpec((_RB, 32), lambda i: (i, 0))],
        out_shape=(jax.ShapeDtypeStruct((N, 32), jnp.float32),
                   jax.ShapeDtypeStruct((N, 32), jnp.float32)),
    )(h_movie, h_director, h_actor, wpm, wpd, wpa, w1)


def _relu_mm_split(lo, hi, b1, w2):
    nb = N // _RB
    rp = _RB // 4

    eye4 = jnp.eye(4, dtype=jnp.float32)
    b4lo = jnp.tile(b1[:32], 4).reshape(1, 128)
    b4hi = jnp.tile(b1[32:], 4).reshape(1, 128)
    bd_ll = jnp.kron(eye4, w2[:32, :32])
    bd_hl = jnp.kron(eye4, w2[32:, :32])
    bd_lh = jnp.kron(eye4, w2[:32, 32:])
    bd_hh = jnp.kron(eye4, w2[32:, 32:])

    def kern(lo_ref, hi_ref, blo_ref, bhi_ref, wll_ref, whl_ref, wlh_ref,
             whh_ref, olo_ref, ohi_ref):
        tlo = jnp.maximum(lo_ref[...].reshape(rp, 128) + blo_ref[...], 0.0)
        thi = jnp.maximum(hi_ref[...].reshape(rp, 128) + bhi_ref[...], 0.0)
        ylo = (jnp.dot(tlo, wll_ref[...], preferred_element_type=jnp.float32)
               + jnp.dot(thi, whl_ref[...], preferred_element_type=jnp.float32))
        yhi = (jnp.dot(tlo, wlh_ref[...], preferred_element_type=jnp.float32)
               + jnp.dot(thi, whh_ref[...], preferred_element_type=jnp.float32))
        olo_ref[...] = ylo.reshape(1, rp, 128)
        ohi_ref[...] = yhi.reshape(1, rp, 128)

    olo, ohi = pl.pallas_call(
        kern,
        grid=(nb,),
        in_specs=[pl.BlockSpec((1, rp, 128), lambda i: (i, 0, 0)),
                  pl.BlockSpec((1, rp, 128), lambda i: (i, 0, 0)),
                  pl.BlockSpec((1, 128), lambda i: (0, 0)),
                  pl.BlockSpec((1, 128), lambda i: (0, 0)),
                  pl.BlockSpec((128, 128), lambda i: (0, 0)),
                  pl.BlockSpec((128, 128), lambda i: (0, 0)),
                  pl.BlockSpec((128, 128), lambda i: (0, 0)),
                  pl.BlockSpec((128, 128), lambda i: (0, 0))],
        out_specs=[pl.BlockSpec((1, rp, 128), lambda i: (i, 0, 0)),
                   pl.BlockSpec((1, rp, 128), lambda i: (i, 0, 0))],
        out_shape=(jax.ShapeDtypeStruct((nb, rp, 128), jnp.float32),
                   jax.ShapeDtypeStruct((nb, rp, 128), jnp.float32)),
    )(lo.reshape(nb, rp, 128), hi.reshape(nb, rp, 128),
      b4lo, b4hi, bd_ll, bd_hl, bd_lh, bd_hh)
    return olo.reshape(N, 32), ohi.reshape(N, 32)


def _norm_abc(lo, hi, b2, w_ctx_d, w_ctx_a, w_hid):
    nb_m = N_M // _RB
    nb_md = (N_M + N_D) // _RB

    def kern(lo_ref, hi_ref, b_ref, wd_ref, wa_ref, wh_ref, h_ref, abc_ref):
        i = pl.program_id(0)
        x = jnp.concatenate([lo_ref[...], hi_ref[...]], axis=1)
        t = x + b_ref[...]
        n = jnp.sqrt(jnp.sum(t * t, axis=1, keepdims=True))
        h = t / jnp.maximum(n, 1e-12)
        h_ref[...] = h
        wh = wh_ref[...]
        wm = wh[:64, :]
        wd = jnp.dot(wd_ref[...], wh[64:80, :], preferred_element_type=jnp.float32)
        wa = jnp.dot(wa_ref[...], wh[80:96, :], preferred_element_type=jnp.float32)
        w = jnp.where(i < nb_m, wm, jnp.where(i < nb_md, wd, wa))
        abc_ref[...] = jnp.dot(h, w, preferred_element_type=jnp.float32)

    return pl.pallas_call(
        kern,
        grid=(N // _RB,),
        in_specs=[pl.BlockSpec((_RB, 32), lambda i: (i, 0)),
                  pl.BlockSpec((_RB, 32), lambda i: (i, 0)),
                  pl.BlockSpec((1, 64), lambda i: (0, 0)),
                  pl.BlockSpec((64, 16), lambda i: (0, 0)),
                  pl.BlockSpec((64, 16), lambda i: (0, 0)),
                  pl.BlockSpec((96, 16), lambda i: (0, 0))],
        out_specs=[pl.BlockSpec((_RB, 64), lambda i: (i, 0)),
                   pl.BlockSpec((_RB, 16), lambda i: (i, 0))],
        out_shape=(jax.ShapeDtypeStruct((N, 64), jnp.float32),
                   jax.ShapeDtypeStruct((N, 16), jnp.float32)),
    )(lo, hi, b2.reshape(1, 64), w_ctx_d, w_ctx_a, w_hid)


def kernel(h_movie, h_director, h_actor, edge_index, ns1_movie, ns1_director,
           ns1_actor, ns2_movie, ns2_director, ns2_actor, W_proj_movie,
           W_proj_director, W_proj_actor, W_gnn1, b_gnn1, W_gnn2, b_gnn2,
           W_ctx_movie, W_ctx_director, W_ctx_actor, W_hid, W_out):
    lo0, hi0 = _proj_fused(h_movie, h_director, h_actor,
                           W_proj_movie, W_proj_director, W_proj_actor, W_gnn1)

    zblk = jnp.zeros((ROWS_PER_TILE, 32), jnp.float32)

    a1lo, a1hi = _sc_agg(edge_index, lo0, hi0, zblk)
    h1lo, h1hi = _relu_mm_split(a1lo, a1hi, b_gnn1, W_gnn2)
    a2lo, a2hi = _sc_agg(edge_index, h1lo, h1hi, zblk)
    h, abc = _norm_abc(a2lo, a2hi, b_gnn2, W_ctx_director, W_ctx_actor, W_hid)

    spad = SP - S_ALL
    zpad = jnp.zeros((spad,), jnp.int32)
    im = jnp.concatenate([ns1_movie, ns2_movie, zpad])
    idd = jnp.concatenate([ns1_director, ns2_director, zpad])
    ia = jnp.concatenate([ns1_actor, ns2_actor, zpad])

    x_full = _sc_tail(abc, im, idd, ia, W_out.reshape(16))
    x = x_full[:S_ALL]

    out_h = h[4353:8029]
    return (h, x, out_h)


# speedup vs baseline: 1.7511x; 1.0266x over previous
import functools

import jax
import jax.numpy as jnp
from jax import lax
from jax.experimental import pallas as pl
from jax.experimental.pallas import tpu as pltpu
from jax.experimental.pallas import tpu_sc as plsc

N_M = 20000
N_D = 10000
N_A = 20000
N = N_M + N_D + N_A
E = 800000

CHUNK = 128
STREAMS = 6
N_TILES = 16
N_STREAMS = E // CHUNK
NS_BASE = N_STREAMS // N_TILES
BURSTS = NS_BASE // STREAMS

NROW_PAD = 50048
ROWS_PER_TILE = NROW_PAD // N_TILES
OUT_ROWS_PER_TILE = N // N_TILES

S_ALL = 40000
SP = 40960
TAIL_CHUNKS_PER_W = SP // (32 * CHUNK)

_mesh = plsc.VectorSubcoreMesh(core_axis_name="c", subcore_axis_name="s")


@functools.partial(
    pl.kernel,
    out_type=(
        jax.ShapeDtypeStruct((N, 32), jnp.float32),
        jax.ShapeDtypeStruct((N, 32), jnp.float32),
    ),
    mesh=_mesh,
    scratch_types=[
        pltpu.VMEM_SHARED((NROW_PAD, 32), jnp.float32),
        pltpu.VMEM((STREAMS, CHUNK), jnp.int32),
        pltpu.VMEM((STREAMS, CHUNK), jnp.int32),
        pltpu.VMEM((STREAMS * CHUNK, 32), jnp.float32),
        pltpu.SemaphoreType.DMA,
        pltpu.SemaphoreType.DMA,
        pltpu.SemaphoreType.DMA,
    ],
    compiler_params=pltpu.CompilerParams(use_tc_tiling_on_sc=False),
)
def _sc_agg(ei, h_lo, h_hi, zblk, out_lo, out_hi,
            acc, srcv, dstv, rows, isem, gsem, ssem):
    c = lax.axis_index("c")
    s = lax.axis_index("s")

    pltpu.sync_copy(zblk, acc.at[pl.ds(s * ROWS_PER_TILE, ROWS_PER_TILE)])
    plsc.subcore_barrier()

    first = s * NS_BASE + jnp.minimum(s, 10)
    extra = s < 10

    def run(h_hbm, out_hbm):
        def do_streams(base_stream, trange):
            descs = []
            for t in trange:
                cb = (base_stream + t) * CHUNK
                descs.append(pltpu.async_copy(
                    ei.at[0, pl.ds(cb, CHUNK)], srcv.at[t], isem))
                descs.append(pltpu.async_copy(
                    ei.at[1, pl.ds(cb, CHUNK)], dstv.at[t], isem))
            for d in descs:
                d.wait()
            descs = []
            for t in trange:
                descs.append(pltpu.async_copy(
                    h_hbm.at[srcv.at[t]],
                    rows.at[pl.ds(t * CHUNK, CHUNK)], gsem))
            for d in descs:
                d.wait()
            descs = []
            for t in trange:
                descs.append(pltpu.async_copy(
                    rows.at[pl.ds(t * CHUNK, CHUNK)],
                    acc.at[dstv.at[t]], ssem, add=True))
            for d in descs:
                d.wait()

        def body(j, carry):
            do_streams(first + j * STREAMS, range(STREAMS))
            return carry

        lax.fori_loop(0, BURSTS, body, 0)

        @pl.when(extra)
        def _():
            do_streams(first + NS_BASE, range(1))

        plsc.subcore_barrier()
        pltpu.sync_copy(acc.at[pl.ds(s * OUT_ROWS_PER_TILE, OUT_ROWS_PER_TILE)],
                        out_hbm.at[pl.ds(s * OUT_ROWS_PER_TILE, OUT_ROWS_PER_TILE)])

    @pl.when(c == 0)
    def _():
        run(h_lo, out_lo)

    @pl.when(c == 1)
    def _():
        run(h_hi, out_hi)


@functools.partial(
    pl.kernel,
    out_type=jax.ShapeDtypeStruct((SP,), jnp.float32),
    mesh=_mesh,
    scratch_types=[
        pltpu.VMEM((CHUNK,), jnp.int32),
        pltpu.VMEM((CHUNK,), jnp.int32),
        pltpu.VMEM((CHUNK,), jnp.int32),
        pltpu.VMEM((CHUNK, 16), jnp.float32),
        pltpu.VMEM((CHUNK, 16), jnp.float32),
        pltpu.VMEM((CHUNK, 16), jnp.float32),
        pltpu.VMEM((CHUNK, 16), jnp.float32),
        pltpu.VMEM((CHUNK,), jnp.float32),
        pltpu.VMEM((16,), jnp.float32),
        pltpu.SemaphoreType.DMA,
    ],
    compiler_params=pltpu.CompilerParams(use_tc_tiling_on_sc=False,
                                         needs_layout_passes=False),
)
def _sc_tail(abc_hbm, im_hbm, id_hbm, ia_hbm, w_hbm, x_hbm,
             imv, idv, iav, ra, rb, rc, tb, xv, wbuf, sem):
    c = lax.axis_index("c")
    s = lax.axis_index("s")
    w = s * 2 + c

    pltpu.sync_copy(w_hbm, wbuf)
    wv = wbuf[...]

    def body(j, carry):
        base = (w * TAIL_CHUNKS_PER_W + j) * CHUNK
        pltpu.sync_copy(im_hbm.at[pl.ds(base, CHUNK)], imv)
        pltpu.sync_copy(id_hbm.at[pl.ds(base, CHUNK)], idv)
        pltpu.sync_copy(ia_hbm.at[pl.ds(base, CHUNK)], iav)
        for u in range(CHUNK // 16):
            sl = pl.ds(u * 16, 16)
            idv[sl] = idv[sl] + N_M
            iav[sl] = iav[sl] + (N_M + N_D)
        d1 = pltpu.async_copy(abc_hbm.at[imv], ra, sem)
        d2 = pltpu.async_copy(abc_hbm.at[idv], rb, sem)
        d3 = pltpu.async_copy(abc_hbm.at[iav], rc, sem)
        d1.wait()
        d2.wait()
        d3.wait()

        def inner(i, carry2):
            row = jnp.maximum(ra[i, :] + rb[i, :] + rc[i, :], 0.0)
            tb[i, :] = row * wv
            return carry2

        lax.fori_loop(0, CHUNK, inner, 0)
        for u in range(CHUNK // 16):
            rowid = jax.lax.iota(jnp.int32, 16) + (u * 16)
            p = jnp.zeros((16,), jnp.float32)
            for f in range(16):
                p = p + plsc.load_gather(
                    tb, [rowid, jnp.full((16,), f, jnp.int32)])
            xv[pl.ds(u * 16, 16)] = 1.0 / (1.0 + jnp.exp(-p))
        pltpu.sync_copy(xv, x_hbm.at[pl.ds(base, CHUNK)])
        return carry

    lax.fori_loop(0, TAIL_CHUNKS_PER_W, body, 0)


_RB = 5000


def _proj_fused(h_movie, h_director, h_actor, wpm, wpd, wpa, w1):
    nb_m = N_M // _RB
    nb_md = (N_M + N_D) // _RB

    def kern(xm_ref, xd_ref, xa_ref, wpm_ref, wpd_ref, wpa_ref, w1_ref,
             lo_ref, hi_ref):
        i = pl.program_id(0)
        w1 = w1_ref[...]
        ym = jnp.dot(xm_ref[...], jnp.dot(wpm_ref[...], w1,
                                          preferred_element_type=jnp.float32),
                     preferred_element_type=jnp.float32)
        wda = jnp.where(i < nb_md, wpd_ref[...], wpa_ref[...])
        xda = jnp.where(i < nb_md, xd_ref[...], xa_ref[...])
        yda = jnp.dot(xda, jnp.dot(wda, w1, preferred_element_type=jnp.float32),
                      preferred_element_type=jnp.float32)
        y = jnp.where(i < nb_m, ym, yda)
        lo_ref[...] = y[:, :32]
        hi_ref[...] = y[:, 32:]

    return pl.pallas_call(
        kern,
        grid=(N // _RB,),
        in_specs=[
            pl.BlockSpec((_RB, 128), lambda i: (jnp.minimum(i, nb_m - 1), 0)),
            pl.BlockSpec((_RB, 64),
                         lambda i: (jnp.clip(i - nb_m, 0, N_D // _RB - 1), 0)),
            pl.BlockSpec((_RB, 64),
                         lambda i: (jnp.clip(i - nb_md, 0, N_A // _RB - 1), 0)),
            pl.BlockSpec((128, 64), lambda i: (0, 0)),
            pl.BlockSpec((64, 64), lambda i: (0, 0)),
            pl.BlockSpec((64, 64), lambda i: (0, 0)),
            pl.BlockSpec((64, 64), lambda i: (0, 0)),
        ],
        out_specs=[pl.BlockSpec((_RB, 32), lambda i: (i, 0)),
                   pl.BlockSpec((_RB, 32), lambda i: (i, 0))],
        out_shape=(jax.ShapeDtypeStruct((N, 32), jnp.float32),
                   jax.ShapeDtypeStruct((N, 32), jnp.float32)),
    )(h_movie, h_director, h_actor, wpm, wpd, wpa, w1)


def _relu_mm_split(lo, hi, b1, w2):
    nb = N // _RB
    rp = _RB // 4

    eye4 = jnp.eye(4, dtype=jnp.float32)
    b4lo = jnp.tile(b1[:32], 4).reshape(1, 128)
    b4hi = jnp.tile(b1[32:], 4).reshape(1, 128)
    bd_ll = jnp.kron(eye4, w2[:32, :32])
    bd_hl = jnp.kron(eye4, w2[32:, :32])
    bd_lh = jnp.kron(eye4, w2[:32, 32:])
    bd_hh = jnp.kron(eye4, w2[32:, 32:])

    def kern(lo_ref, hi_ref, blo_ref, bhi_ref, wll_ref, whl_ref, wlh_ref,
             whh_ref, olo_ref, ohi_ref):
        tlo = jnp.maximum(lo_ref[...].reshape(rp, 128) + blo_ref[...], 0.0)
        thi = jnp.maximum(hi_ref[...].reshape(rp, 128) + bhi_ref[...], 0.0)
        ylo = (jnp.dot(tlo, wll_ref[...], preferred_element_type=jnp.float32)
               + jnp.dot(thi, whl_ref[...], preferred_element_type=jnp.float32))
        yhi = (jnp.dot(tlo, wlh_ref[...], preferred_element_type=jnp.float32)
               + jnp.dot(thi, whh_ref[...], preferred_element_type=jnp.float32))
        olo_ref[...] = ylo.reshape(1, rp, 128)
        ohi_ref[...] = yhi.reshape(1, rp, 128)

    olo, ohi = pl.pallas_call(
        kern,
        grid=(nb,),
        in_specs=[pl.BlockSpec((1, rp, 128), lambda i: (i, 0, 0)),
                  pl.BlockSpec((1, rp, 128), lambda i: (i, 0, 0)),
                  pl.BlockSpec((1, 128), lambda i: (0, 0)),
                  pl.BlockSpec((1, 128), lambda i: (0, 0)),
                  pl.BlockSpec((128, 128), lambda i: (0, 0)),
                  pl.BlockSpec((128, 128), lambda i: (0, 0)),
                  pl.BlockSpec((128, 128), lambda i: (0, 0)),
                  pl.BlockSpec((128, 128), lambda i: (0, 0))],
        out_specs=[pl.BlockSpec((1, rp, 128), lambda i: (i, 0, 0)),
                   pl.BlockSpec((1, rp, 128), lambda i: (i, 0, 0))],
        out_shape=(jax.ShapeDtypeStruct((nb, rp, 128), jnp.float32),
                   jax.ShapeDtypeStruct((nb, rp, 128), jnp.float32)),
    )(lo.reshape(nb, rp, 128), hi.reshape(nb, rp, 128),
      b4lo, b4hi, bd_ll, bd_hl, bd_lh, bd_hh)
    return olo.reshape(N, 32), ohi.reshape(N, 32)


def _norm_abc(lo, hi, b2, w_ctx_d, w_ctx_a, w_hid):
    nb_m = N_M // _RB
    nb_md = (N_M + N_D) // _RB

    def kern(lo_ref, hi_ref, b_ref, wd_ref, wa_ref, wh_ref, h_ref, abc_ref):
        i = pl.program_id(0)
        x = jnp.concatenate([lo_ref[...], hi_ref[...]], axis=1)
        t = x + b_ref[...]
        n = jnp.sqrt(jnp.sum(t * t, axis=1, keepdims=True))
        h = t / jnp.maximum(n, 1e-12)
        h_ref[...] = h
        wh = wh_ref[...]
        wm = wh[:64, :]
        wd = jnp.dot(wd_ref[...], wh[64:80, :], preferred_element_type=jnp.float32)
        wa = jnp.dot(wa_ref[...], wh[80:96, :], preferred_element_type=jnp.float32)
        w = jnp.where(i < nb_m, wm, jnp.where(i < nb_md, wd, wa))
        abc_ref[...] = jnp.dot(h, w, preferred_element_type=jnp.float32)

    return pl.pallas_call(
        kern,
        grid=(N // _RB,),
        in_specs=[pl.BlockSpec((_RB, 32), lambda i: (i, 0)),
                  pl.BlockSpec((_RB, 32), lambda i: (i, 0)),
                  pl.BlockSpec((1, 64), lambda i: (0, 0)),
                  pl.BlockSpec((64, 16), lambda i: (0, 0)),
                  pl.BlockSpec((64, 16), lambda i: (0, 0)),
                  pl.BlockSpec((96, 16), lambda i: (0, 0))],
        out_specs=[pl.BlockSpec((_RB, 64), lambda i: (i, 0)),
                   pl.BlockSpec((_RB, 16), lambda i: (i, 0))],
        out_shape=(jax.ShapeDtypeStruct((N, 64), jnp.float32),
                   jax.ShapeDtypeStruct((N, 16), jnp.float32)),
    )(lo, hi, b2.reshape(1, 64), w_ctx_d, w_ctx_a, w_hid)


def kernel(h_movie, h_director, h_actor, edge_index, ns1_movie, ns1_director,
           ns1_actor, ns2_movie, ns2_director, ns2_actor, W_proj_movie,
           W_proj_director, W_proj_actor, W_gnn1, b_gnn1, W_gnn2, b_gnn2,
           W_ctx_movie, W_ctx_director, W_ctx_actor, W_hid, W_out):
    lo0, hi0 = _proj_fused(h_movie, h_director, h_actor,
                           W_proj_movie, W_proj_director, W_proj_actor, W_gnn1)

    zblk = jnp.zeros((ROWS_PER_TILE, 32), jnp.float32)

    a1lo, a1hi = _sc_agg(edge_index, lo0, hi0, zblk)
    h1lo, h1hi = _relu_mm_split(a1lo, a1hi, b_gnn1, W_gnn2)
    a2lo, a2hi = _sc_agg(edge_index, h1lo, h1hi, zblk)
    h, abc = _norm_abc(a2lo, a2hi, b_gnn2, W_ctx_director, W_ctx_actor, W_hid)

    spad = SP - S_ALL
    zpad = jnp.zeros((spad,), jnp.int32)
    im = jnp.concatenate([ns1_movie, ns2_movie, zpad])
    idd = jnp.concatenate([ns1_director, ns2_director, zpad])
    ia = jnp.concatenate([ns1_actor, ns2_actor, zpad])

    x_full = _sc_tail(abc, im, idd, ia, W_out.reshape(16))
    x = x_full[:S_ALL]

    out_h = h[4353:8029]
    return (h, x, out_h)


# speedup vs baseline: 1.7851x; 1.0194x over previous
import functools

import jax
import jax.numpy as jnp
from jax import lax
from jax.experimental import pallas as pl
from jax.experimental.pallas import tpu as pltpu
from jax.experimental.pallas import tpu_sc as plsc

N_M = 20000
N_D = 10000
N_A = 20000
N = N_M + N_D + N_A
E = 800000

CHUNK = 128
STREAMS = 6
N_TILES = 16
N_STREAMS = E // CHUNK
NS_BASE = N_STREAMS // N_TILES
BURSTS = NS_BASE // STREAMS

NROW_PAD = 50048
ROWS_PER_TILE = NROW_PAD // N_TILES
OUT_ROWS_PER_TILE = N // N_TILES

S_ALL = 40000
SP = 40960
TAIL_CHUNKS_PER_W = SP // (32 * CHUNK)

_mesh = plsc.VectorSubcoreMesh(core_axis_name="c", subcore_axis_name="s")


@functools.partial(
    pl.kernel,
    out_type=(
        jax.ShapeDtypeStruct((N, 32), jnp.float32),
        jax.ShapeDtypeStruct((N, 32), jnp.float32),
    ),
    mesh=_mesh,
    scratch_types=[
        pltpu.VMEM_SHARED((NROW_PAD, 32), jnp.float32),
        pltpu.VMEM((STREAMS, CHUNK), jnp.int32),
        pltpu.VMEM((STREAMS, CHUNK), jnp.int32),
        pltpu.VMEM((STREAMS * CHUNK, 32), jnp.float32),
        pltpu.SemaphoreType.DMA,
        pltpu.SemaphoreType.DMA,
        pltpu.SemaphoreType.DMA,
    ],
    compiler_params=pltpu.CompilerParams(use_tc_tiling_on_sc=False),
)
def _sc_agg(ei, h_lo, h_hi, zblk, out_lo, out_hi,
            acc, srcv, dstv, rows, isem, gsem, ssem):
    c = lax.axis_index("c")
    s = lax.axis_index("s")

    pltpu.sync_copy(zblk, acc.at[pl.ds(s * ROWS_PER_TILE, ROWS_PER_TILE)])
    plsc.subcore_barrier()

    first = s * NS_BASE + jnp.minimum(s, 10)
    extra = s < 10

    def run(h_hbm, out_hbm):
        def do_streams(base_stream, trange):
            descs = []
            for t in trange:
                cb = (base_stream + t) * CHUNK
                descs.append(pltpu.async_copy(
                    ei.at[0, pl.ds(cb, CHUNK)], srcv.at[t], isem))
                descs.append(pltpu.async_copy(
                    ei.at[1, pl.ds(cb, CHUNK)], dstv.at[t], isem))
            for d in descs:
                d.wait()
            descs = []
            for t in trange:
                descs.append(pltpu.async_copy(
                    h_hbm.at[srcv.at[t]],
                    rows.at[pl.ds(t * CHUNK, CHUNK)], gsem))
            for d in descs:
                d.wait()
            descs = []
            for t in trange:
                descs.append(pltpu.async_copy(
                    rows.at[pl.ds(t * CHUNK, CHUNK)],
                    acc.at[dstv.at[t]], ssem, add=True))
            for d in descs:
                d.wait()

        def body(j, carry):
            do_streams(first + j * STREAMS, range(STREAMS))
            return carry

        lax.fori_loop(0, BURSTS, body, 0)

        @pl.when(extra)
        def _():
            do_streams(first + NS_BASE, range(1))

        plsc.subcore_barrier()
        pltpu.sync_copy(acc.at[pl.ds(s * OUT_ROWS_PER_TILE, OUT_ROWS_PER_TILE)],
                        out_hbm.at[pl.ds(s * OUT_ROWS_PER_TILE, OUT_ROWS_PER_TILE)])

    @pl.when(c == 0)
    def _():
        run(h_lo, out_lo)

    @pl.when(c == 1)
    def _():
        run(h_hi, out_hi)


@functools.partial(
    pl.kernel,
    out_type=jax.ShapeDtypeStruct((SP,), jnp.float32),
    mesh=_mesh,
    scratch_types=[
        pltpu.VMEM((CHUNK,), jnp.int32),
        pltpu.VMEM((CHUNK,), jnp.int32),
        pltpu.VMEM((CHUNK,), jnp.int32),
        pltpu.VMEM((CHUNK, 16), jnp.float32),
        pltpu.VMEM((CHUNK, 16), jnp.float32),
        pltpu.VMEM((CHUNK, 16), jnp.float32),
        pltpu.VMEM((CHUNK, 16), jnp.float32),
        pltpu.VMEM((CHUNK,), jnp.float32),
        pltpu.VMEM((16,), jnp.float32),
        pltpu.SemaphoreType.DMA,
    ],
    compiler_params=pltpu.CompilerParams(use_tc_tiling_on_sc=False,
                                         needs_layout_passes=False),
)
def _sc_tail(abc_hbm, im_hbm, id_hbm, ia_hbm, w_hbm, x_hbm,
             imv, idv, iav, ra, rb, rc, tb, xv, wbuf, sem):
    c = lax.axis_index("c")
    s = lax.axis_index("s")
    w = s * 2 + c

    pltpu.sync_copy(w_hbm, wbuf)
    wv = wbuf[...]

    def body(j, carry):
        base = (w * TAIL_CHUNKS_PER_W + j) * CHUNK
        pltpu.sync_copy(im_hbm.at[pl.ds(base, CHUNK)], imv)
        pltpu.sync_copy(id_hbm.at[pl.ds(base, CHUNK)], idv)
        pltpu.sync_copy(ia_hbm.at[pl.ds(base, CHUNK)], iav)
        for u in range(CHUNK // 16):
            sl = pl.ds(u * 16, 16)
            idv[sl] = idv[sl] + N_M
            iav[sl] = iav[sl] + (N_M + N_D)
        d1 = pltpu.async_copy(abc_hbm.at[imv], ra, sem)
        d2 = pltpu.async_copy(abc_hbm.at[idv], rb, sem)
        d3 = pltpu.async_copy(abc_hbm.at[iav], rc, sem)
        d1.wait()
        d2.wait()
        d3.wait()

        def inner(i, carry2):
            row = jnp.maximum(ra[i, :] + rb[i, :] + rc[i, :], 0.0)
            tb[i, :] = row * wv
            return carry2

        lax.fori_loop(0, CHUNK, inner, 0)
        for u in range(CHUNK // 16):
            rowid = jax.lax.iota(jnp.int32, 16) + (u * 16)
            p = jnp.zeros((16,), jnp.float32)
            for f in range(16):
                p = p + plsc.load_gather(
                    tb, [rowid, jnp.full((16,), f, jnp.int32)])
            xv[pl.ds(u * 16, 16)] = 1.0 / (1.0 + jnp.exp(-p))
        pltpu.sync_copy(xv, x_hbm.at[pl.ds(base, CHUNK)])
        return carry

    lax.fori_loop(0, TAIL_CHUNKS_PER_W, body, 0)


_RB = 10000


def _proj_fused(h_movie, h_director, h_actor, wpm, wpd, wpa, w1):
    nb_m = N_M // _RB
    nb_md = (N_M + N_D) // _RB

    def kern(xm_ref, xd_ref, xa_ref, wpm_ref, wpd_ref, wpa_ref, w1_ref,
             lo_ref, hi_ref):
        i = pl.program_id(0)
        w1 = w1_ref[...]
        ym = jnp.dot(xm_ref[...], jnp.dot(wpm_ref[...], w1,
                                          preferred_element_type=jnp.float32),
                     preferred_element_type=jnp.float32)
        wda = jnp.where(i < nb_md, wpd_ref[...], wpa_ref[...])
        xda = jnp.where(i < nb_md, xd_ref[...], xa_ref[...])
        yda = jnp.dot(xda, jnp.dot(wda, w1, preferred_element_type=jnp.float32),
                      preferred_element_type=jnp.float32)
        y = jnp.where(i < nb_m, ym, yda)
        lo_ref[...] = y[:, :32]
        hi_ref[...] = y[:, 32:]

    return pl.pallas_call(
        kern,
        grid=(N // _RB,),
        in_specs=[
            pl.BlockSpec((_RB, 128), lambda i: (jnp.minimum(i, nb_m - 1), 0)),
            pl.BlockSpec((_RB, 64),
                         lambda i: (jnp.clip(i - nb_m, 0, N_D // _RB - 1), 0)),
            pl.BlockSpec((_RB, 64),
                         lambda i: (jnp.clip(i - nb_md, 0, N_A // _RB - 1), 0)),
            pl.BlockSpec((128, 64), lambda i: (0, 0)),
            pl.BlockSpec((64, 64), lambda i: (0, 0)),
            pl.BlockSpec((64, 64), lambda i: (0, 0)),
            pl.BlockSpec((64, 64), lambda i: (0, 0)),
        ],
        out_specs=[pl.BlockSpec((_RB, 32), lambda i: (i, 0)),
                   pl.BlockSpec((_RB, 32), lambda i: (i, 0))],
        out_shape=(jax.ShapeDtypeStruct((N, 32), jnp.float32),
                   jax.ShapeDtypeStruct((N, 32), jnp.float32)),
    )(h_movie, h_director, h_actor, wpm, wpd, wpa, w1)


def _relu_mm_split(lo, hi, b1, w2):
    nb = N // _RB
    rp = _RB // 4

    eye4 = jnp.eye(4, dtype=jnp.float32)
    b4lo = jnp.tile(b1[:32], 4).reshape(1, 128)
    b4hi = jnp.tile(b1[32:], 4).reshape(1, 128)
    bd_ll = jnp.kron(eye4, w2[:32, :32])
    bd_hl = jnp.kron(eye4, w2[32:, :32])
    bd_lh = jnp.kron(eye4, w2[:32, 32:])
    bd_hh = jnp.kron(eye4, w2[32:, 32:])

    def kern(lo_ref, hi_ref, blo_ref, bhi_ref, wll_ref, whl_ref, wlh_ref,
             whh_ref, olo_ref, ohi_ref):
        tlo = jnp.maximum(lo_ref[...].reshape(rp, 128) + blo_ref[...], 0.0)
        thi = jnp.maximum(hi_ref[...].reshape(rp, 128) + bhi_ref[...], 0.0)
        ylo = (jnp.dot(tlo, wll_ref[...], preferred_element_type=jnp.float32)
               + jnp.dot(thi, whl_ref[...], preferred_element_type=jnp.float32))
        yhi = (jnp.dot(tlo, wlh_ref[...], preferred_element_type=jnp.float32)
               + jnp.dot(thi, whh_ref[...], preferred_element_type=jnp.float32))
        olo_ref[...] = ylo.reshape(1, rp, 128)
        ohi_ref[...] = yhi.reshape(1, rp, 128)

    olo, ohi = pl.pallas_call(
        kern,
        grid=(nb,),
        in_specs=[pl.BlockSpec((1, rp, 128), lambda i: (i, 0, 0)),
                  pl.BlockSpec((1, rp, 128), lambda i: (i, 0, 0)),
                  pl.BlockSpec((1, 128), lambda i: (0, 0)),
                  pl.BlockSpec((1, 128), lambda i: (0, 0)),
                  pl.BlockSpec((128, 128), lambda i: (0, 0)),
                  pl.BlockSpec((128, 128), lambda i: (0, 0)),
                  pl.BlockSpec((128, 128), lambda i: (0, 0)),
                  pl.BlockSpec((128, 128), lambda i: (0, 0))],
        out_specs=[pl.BlockSpec((1, rp, 128), lambda i: (i, 0, 0)),
                   pl.BlockSpec((1, rp, 128), lambda i: (i, 0, 0))],
        out_shape=(jax.ShapeDtypeStruct((nb, rp, 128), jnp.float32),
                   jax.ShapeDtypeStruct((nb, rp, 128), jnp.float32)),
    )(lo.reshape(nb, rp, 128), hi.reshape(nb, rp, 128),
      b4lo, b4hi, bd_ll, bd_hl, bd_lh, bd_hh)
    return olo.reshape(N, 32), ohi.reshape(N, 32)


def _norm_abc(lo, hi, b2, w_ctx_d, w_ctx_a, w_hid):
    nb_m = N_M // _RB
    nb_md = (N_M + N_D) // _RB

    def kern(lo_ref, hi_ref, b_ref, wd_ref, wa_ref, wh_ref, h_ref, abc_ref):
        i = pl.program_id(0)
        x = jnp.concatenate([lo_ref[...], hi_ref[...]], axis=1)
        t = x + b_ref[...]
        n = jnp.sqrt(jnp.sum(t * t, axis=1, keepdims=True))
        h = t / jnp.maximum(n, 1e-12)
        h_ref[...] = h
        wh = wh_ref[...]
        wm = wh[:64, :]
        wd = jnp.dot(wd_ref[...], wh[64:80, :], preferred_element_type=jnp.float32)
        wa = jnp.dot(wa_ref[...], wh[80:96, :], preferred_element_type=jnp.float32)
        w = jnp.where(i < nb_m, wm, jnp.where(i < nb_md, wd, wa))
        abc_ref[...] = jnp.dot(h, w, preferred_element_type=jnp.float32)

    return pl.pallas_call(
        kern,
        grid=(N // _RB,),
        in_specs=[pl.BlockSpec((_RB, 32), lambda i: (i, 0)),
                  pl.BlockSpec((_RB, 32), lambda i: (i, 0)),
                  pl.BlockSpec((1, 64), lambda i: (0, 0)),
                  pl.BlockSpec((64, 16), lambda i: (0, 0)),
                  pl.BlockSpec((64, 16), lambda i: (0, 0)),
                  pl.BlockSpec((96, 16), lambda i: (0, 0))],
        out_specs=[pl.BlockSpec((_RB, 64), lambda i: (i, 0)),
                   pl.BlockSpec((_RB, 16), lambda i: (i, 0))],
        out_shape=(jax.ShapeDtypeStruct((N, 64), jnp.float32),
                   jax.ShapeDtypeStruct((N, 16), jnp.float32)),
    )(lo, hi, b2.reshape(1, 64), w_ctx_d, w_ctx_a, w_hid)


def kernel(h_movie, h_director, h_actor, edge_index, ns1_movie, ns1_director,
           ns1_actor, ns2_movie, ns2_director, ns2_actor, W_proj_movie,
           W_proj_director, W_proj_actor, W_gnn1, b_gnn1, W_gnn2, b_gnn2,
           W_ctx_movie, W_ctx_director, W_ctx_actor, W_hid, W_out):
    lo0, hi0 = _proj_fused(h_movie, h_director, h_actor,
                           W_proj_movie, W_proj_director, W_proj_actor, W_gnn1)

    zblk = jnp.zeros((ROWS_PER_TILE, 32), jnp.float32)

    a1lo, a1hi = _sc_agg(edge_index, lo0, hi0, zblk)
    h1lo, h1hi = _relu_mm_split(a1lo, a1hi, b_gnn1, W_gnn2)
    a2lo, a2hi = _sc_agg(edge_index, h1lo, h1hi, zblk)
    h, abc = _norm_abc(a2lo, a2hi, b_gnn2, W_ctx_director, W_ctx_actor, W_hid)

    spad = SP - S_ALL
    zpad = jnp.zeros((spad,), jnp.int32)
    im = jnp.concatenate([ns1_movie, ns2_movie, zpad])
    idd = jnp.concatenate([ns1_director, ns2_director, zpad])
    ia = jnp.concatenate([ns1_actor, ns2_actor, zpad])

    x_full = _sc_tail(abc, im, idd, ia, W_out.reshape(16))
    x = x_full[:S_ALL]

    out_h = h[4353:8029]
    return (h, x, out_h)
